# baseline (speedup 1.0000x reference)
import jax
import jax.numpy as jnp
from jax.experimental import pallas as pl
from jax.experimental.pallas import tpu as pltpu

T = 2048
C = 128
K = 3
CYC = 64
NB = 2
EPS_GN = 1e-5


def _gelu(x):
    return 0.5 * x * (1.0 + jnp.tanh(0.7978845608028654 * (x + 0.044715 * x * x * x)))


def _body(x_ref, wp_ref, out_ref):
    W_dw9 = wp_ref[0:9]
    gn_gamma = wp_ref[9:10]
    gn_beta = wp_ref[10:11]
    res_row = wp_ref[11:12]
    W_pwT = wp_ref[16:16 + C]
    W_gateT = wp_ref[16 + C:16 + 2 * C]
    W_loc7 = wp_ref[16 + 2 * C:16 + 9 * C].astype(jnp.bfloat16)

    f32 = jnp.float32
    A = [x_ref[n] for n in range(NB)]

    S = []
    for n in range(NB):
        Apad = jnp.concatenate([A[n], A[n][:128]], axis=0)
        Sa = jnp.zeros((128, 256), f32)
        Sb = jnp.zeros((128, 256), f32)
        for i in range(0, 16, 2):
            Sa = Sa + jax.lax.dot_general(
                A[n][128 * i:128 * (i + 1)], Apad[128 * i:128 * i + 256],
                (((1,), (1,)), ((), ())), preferred_element_type=f32)
            Sb = Sb + jax.lax.dot_general(
                A[n][128 * (i + 1):128 * (i + 2)],
                Apad[128 * (i + 1):128 * (i + 1) + 256],
                (((1,), (1,)), ((), ())), preferred_element_type=f32)
        S.append(Sa + Sb)

    local_raw = []
    for n in range(NB):
        xpadb = jnp.concatenate(
            [jnp.zeros((3, C), jnp.bfloat16), A[n].astype(jnp.bfloat16),
             jnp.zeros((3, C), jnp.bfloat16)], axis=0)
        lp = jnp.zeros((T, C), f32)
        for j in range(7):
            lp = lp + jnp.dot(xpadb[j:j + T], W_loc7[C * j:C * (j + 1)],
                              preferred_element_type=f32)
        local_raw.append(_gelu(lp))

    lag = jax.lax.broadcasted_iota(jnp.int32, (1, 256), 1)
    row = jax.lax.broadcasted_iota(jnp.int32, (128, 256), 0)
    NEG = f32(-3e38)
    ids_n, ws_n, gate_n = [], [], []
    for n in range(NB):
        Sn = S[n]
        for k in range(7):
            s = 1 << k
            rolled = jnp.concatenate([Sn[:, s:], Sn[:, :s]], axis=1)
            Sn = jnp.where((row & s) != 0, rolled, Sn)
        r_full = jnp.sum(Sn, axis=0, keepdims=True) * (1.0 / C)
        rm = jnp.where((lag >= 32) & (lag <= 128), r_full, NEG)
        vs, ids = [], []
        for _ in range(K):
            v = jnp.max(rm, axis=1, keepdims=True)
            i_k = jnp.min(jnp.where(rm >= v, lag, 9999), axis=1, keepdims=True)
            rm = jnp.where(lag == i_k, NEG, rm)
            vs.append(v)
            ids.append(i_k)
        vmax = jnp.maximum(jnp.maximum(vs[0], vs[1]), vs[2])
        es = [jnp.exp(v - vmax) for v in vs]
        ssum = es[0] + es[1] + es[2]
        ws = [e / ssum for e in es]
        H = -(ws[0] * jnp.log(ws[0] + 1e-8) + ws[1] * jnp.log(ws[1] + 1e-8)
              + ws[2] * jnp.log(ws[2] + 1e-8))
        Hmax = jnp.log(f32(K) + 1e-8)
        Gamma = jnp.clip(1.0 - H / (Hmax + 1e-8), 0.0, 1.0)
        ids_n.append(ids)
        ws_n.append(ws)
        gate_n.append(jax.nn.sigmoid(4.0 * (Gamma - 0.5)))

    def _psel(r_i, ids):
        return jnp.where(r_i < CYC, ids[0],
                         jnp.where(r_i < 2 * CYC, ids[1], ids[2]))

    U_n, M_n = [], []
    for n in range(NB):
        ids = ids_n[n]
        r_c = jax.lax.broadcasted_iota(jnp.int32, (K * CYC, 1), 0)
        t_i = jax.lax.broadcasted_iota(jnp.int32, (K * CYC, T), 1)
        p_c1 = _psel(r_c, ids)
        q_c1 = (r_c & (CYC - 1)) * p_c1
        M = ((t_i >= q_c1) & (t_i < q_c1 + p_c1)).astype(f32)
        M_n.append(M)
        rr = jax.lax.broadcasted_iota(jnp.int32, (K * CYC, 128), 0)
        ss = jax.lax.broadcasted_iota(jnp.int32, (K * CYC, 128), 1)
        p_c = _psel(rr, ids)
        tpad_c = (jnp.ceil(2048.0 / p_c.astype(f32)) * p_c.astype(f32)
                  ).astype(jnp.int32)
        q_c = (rr & (CYC - 1)) * p_c
        tt = (2 * (T - 1) - (T - 128)) - ss
        refl = ((tt >= T) & (tt < tpad_c) & (tt >= q_c)
                & (tt < q_c + p_c)).astype(f32)
        U = (jnp.dot(M, A[n], preferred_element_type=f32)
             + jnp.dot(refl, A[n][T - 128:], preferred_element_type=f32))
        U_n.append(U * (1.0 / 128.0))

    g_of_c = jax.lax.broadcasted_iota(jnp.int32, (C, 32), 0) // 4
    g_id = jax.lax.broadcasted_iota(jnp.int32, (C, 32), 1)
    G = (g_of_c == g_id).astype(f32)
    X_n = []
    for n in range(NB):
        U = U_n[n]
        z4 = jnp.zeros((4, C), f32)
        z16 = jnp.zeros((16, C), f32)
        Up = jnp.concatenate(
            [z4, U[0:CYC], z16, U[CYC:2 * CYC], z16, U[2 * CYC:3 * CYC], z4],
            axis=0)
        xc = jnp.zeros((224, C), f32)
        for j in range(9):
            xc = xc + Up[j:j + 224] * W_dw9[j][None, :]
        xp = jnp.dot(xc, W_pwT, preferred_element_type=f32)
        xps = [xp[80 * k:80 * k + CYC] for k in range(K)]
        CS = jnp.concatenate(
            [jnp.sum(s, axis=0, keepdims=True) for s in xps], axis=0)
        MU = jnp.dot(jnp.dot(CS, G), G.T) * (1.0 / 256.0)
        devs = [xps[k] - MU[k:k + 1] for k in range(K)]
        VS = jnp.concatenate(
            [jnp.sum(d * d, axis=0, keepdims=True) for d in devs], axis=0)
        VR = jnp.dot(jnp.dot(VS, G), G.T) * (1.0 / 256.0)
        UB = jnp.concatenate(
            [jnp.sum(U[CYC * k:CYC * (k + 1)], axis=0, keepdims=True)
             for k in range(K)], axis=0) * (1.0 / 64.0)
        GV = jax.nn.sigmoid(jnp.dot(UB, W_gateT))
        xks = []
        for k in range(K):
            xn = devs[k] * jax.lax.rsqrt(VR[k:k + 1] + EPS_GN)
            xn = xn * gn_gamma + gn_beta
            xg = _gelu(xn)
            xks.append(xg * (GV[k:k + 1] * (ws_n[n][k] * res_row)))
        X_n.append(jnp.concatenate(xks, axis=0))

    for n in range(NB):
        periodic = A[n] + jax.lax.dot_general(
            M_n[n], X_n[n], (((0,), (0,)), ((), ())),
            preferred_element_type=f32)
        local = local_raw[n]
        num = jnp.sum(local * periodic, axis=0, keepdims=True)
        den = jnp.sum(periodic * periodic, axis=0, keepdims=True) + 1e-6
        local = local - (num / den) * periodic
        gate = gate_n[n]
        fused = gate * periodic + (1.0 - gate) * 0.3 * local
        out_ref[n] = A[n] + fused


@jax.jit
def kernel(x, W_dw, W_pw, gn_gamma, gn_beta, W_gate, res_gamma, W_loc_dw,
           W_loc_pw):
    B = x.shape[0]
    Wpacked = jnp.concatenate([
        W_dw[:, 0, :].T,
        gn_gamma[None, :],
        gn_beta[None, :],
        res_gamma[:, :, 0, 0],
        jnp.zeros((4, C), jnp.float32),
        W_pw.T,
        W_gate.T,
        (W_loc_dw[:, 0, :].T[:, :, None]
         * W_loc_pw.T[None, :, :]).reshape(7 * C, C),
    ], axis=0)

    return pl.pallas_call(
        _body,
        grid=(B // NB,),
        in_specs=[
            pl.BlockSpec((NB, T, C), lambda b: (b, 0, 0)),
            pl.BlockSpec((16 + 9 * C, C), lambda b: (0, 0)),
        ],
        out_specs=pl.BlockSpec((NB, T, C), lambda b: (b, 0, 0)),
        out_shape=jax.ShapeDtypeStruct((B, T, C), jnp.float32),
        compiler_params=pltpu.CompilerParams(
            dimension_semantics=("parallel",),
            fuse_transposed_lhs_in_matmul=True),
    )(x, Wpacked)


# speedup vs baseline: 1.0591x; 1.0591x over previous
import jax
import jax.numpy as jnp
from jax.experimental import pallas as pl
from jax.experimental.pallas import tpu as pltpu

T = 2048
C = 128
K = 3
CYC = 64
NB = 2
EPS_GN = 1e-5


def _gelu(x):
    return 0.5 * x * (1.0 + jnp.tanh(0.7978845608028654 * (x + 0.044715 * x * x * x)))


def _body(x_ref, wdw_ref, wpwt_ref, gng_ref, gnb_ref, wgt_ref, res_ref,
          wloc_ref, out_ref):
    W_dw9 = wdw_ref[...]
    W_pwT = wpwt_ref[...]
    gn_gamma = gng_ref[...]
    gn_beta = gnb_ref[...]
    W_gateT = wgt_ref[...]
    res_row = res_ref[...]
    W_loc7 = wloc_ref[...]

    f32 = jnp.float32
    A = [x_ref[n] for n in range(NB)]

    S = []
    for n in range(NB):
        Apad = jnp.concatenate([A[n], A[n][:128]], axis=0)
        Sa = jnp.zeros((128, 256), f32)
        Sb = jnp.zeros((128, 256), f32)
        for i in range(0, 16, 2):
            Sa = Sa + jax.lax.dot_general(
                A[n][128 * i:128 * (i + 1)], Apad[128 * i:128 * i + 256],
                (((1,), (1,)), ((), ())), preferred_element_type=f32)
            Sb = Sb + jax.lax.dot_general(
                A[n][128 * (i + 1):128 * (i + 2)],
                Apad[128 * (i + 1):128 * (i + 1) + 256],
                (((1,), (1,)), ((), ())), preferred_element_type=f32)
        S.append(Sa + Sb)

    local_raw = []
    for n in range(NB):
        xpadb = jnp.concatenate(
            [jnp.zeros((3, C), jnp.bfloat16), A[n].astype(jnp.bfloat16),
             jnp.zeros((3, C), jnp.bfloat16)], axis=0)
        lp = jnp.zeros((T, C), f32)
        for j in range(7):
            lp = lp + jnp.dot(xpadb[j:j + T], W_loc7[C * j:C * (j + 1)],
                              preferred_element_type=f32)
        local_raw.append(_gelu(lp))

    lag = jax.lax.broadcasted_iota(jnp.int32, (1, 256), 1)
    row = jax.lax.broadcasted_iota(jnp.int32, (128, 256), 0)
    NEG = f32(-3e38)
    ids_n, ws_n, gate_n = [], [], []
    for n in range(NB):
        Sn = S[n]
        for k in range(7):
            s = 1 << k
            rolled = jnp.concatenate([Sn[:, s:], Sn[:, :s]], axis=1)
            Sn = jnp.where((row & s) != 0, rolled, Sn)
        r_full = jnp.sum(Sn, axis=0, keepdims=True) * (1.0 / C)
        rm = jnp.where((lag >= 32) & (lag <= 128), r_full, NEG)
        vs, ids = [], []
        for _ in range(K):
            v = jnp.max(rm, axis=1, keepdims=True)
            i_k = jnp.min(jnp.where(rm >= v, lag, 9999), axis=1, keepdims=True)
            rm = jnp.where(lag == i_k, NEG, rm)
            vs.append(v)
            ids.append(i_k)
        vmax = jnp.maximum(jnp.maximum(vs[0], vs[1]), vs[2])
        es = [jnp.exp(v - vmax) for v in vs]
        ssum = es[0] + es[1] + es[2]
        ws = [e / ssum for e in es]
        H = -(ws[0] * jnp.log(ws[0] + 1e-8) + ws[1] * jnp.log(ws[1] + 1e-8)
              + ws[2] * jnp.log(ws[2] + 1e-8))
        Hmax = jnp.log(f32(K) + 1e-8)
        Gamma = jnp.clip(1.0 - H / (Hmax + 1e-8), 0.0, 1.0)
        ids_n.append(ids)
        ws_n.append(ws)
        gate_n.append(jax.nn.sigmoid(4.0 * (Gamma - 0.5)))

    def _psel(r_i, ids):
        return jnp.where(r_i < CYC, ids[0],
                         jnp.where(r_i < 2 * CYC, ids[1], ids[2]))

    U_n, M_n = [], []
    for n in range(NB):
        ids = ids_n[n]
        r_c = jax.lax.broadcasted_iota(jnp.int32, (K * CYC, 1), 0)
        t_i = jax.lax.broadcasted_iota(jnp.int32, (K * CYC, T), 1)
        p_c1 = _psel(r_c, ids)
        q_c1 = (r_c & (CYC - 1)) * p_c1
        M = ((t_i >= q_c1) & (t_i < q_c1 + p_c1)).astype(f32)
        M_n.append(M)
        rr = jax.lax.broadcasted_iota(jnp.int32, (K * CYC, 128), 0)
        ss = jax.lax.broadcasted_iota(jnp.int32, (K * CYC, 128), 1)
        p_c = _psel(rr, ids)
        tpad_c = (jnp.ceil(2048.0 / p_c.astype(f32)) * p_c.astype(f32)
                  ).astype(jnp.int32)
        q_c = (rr & (CYC - 1)) * p_c
        tt = (2 * (T - 1) - (T - 128)) - ss
        refl = ((tt >= T) & (tt < tpad_c) & (tt >= q_c)
                & (tt < q_c + p_c)).astype(f32)
        U = (jnp.dot(M, A[n], preferred_element_type=f32)
             + jnp.dot(refl, A[n][T - 128:], preferred_element_type=f32))
        U_n.append(U * (1.0 / 128.0))

    g_of_c = jax.lax.broadcasted_iota(jnp.int32, (C, 32), 0) // 4
    g_id = jax.lax.broadcasted_iota(jnp.int32, (C, 32), 1)
    G = (g_of_c == g_id).astype(f32)
    X_n = []
    for n in range(NB):
        U = U_n[n]
        z4 = jnp.zeros((4, C), f32)
        z16 = jnp.zeros((16, C), f32)
        Up = jnp.concatenate(
            [z4, U[0:CYC], z16, U[CYC:2 * CYC], z16, U[2 * CYC:3 * CYC], z4],
            axis=0)
        xc = jnp.zeros((224, C), f32)
        for j in range(9):
            xc = xc + Up[j:j + 224] * W_dw9[j][None, :]
        xp = jnp.dot(xc, W_pwT, preferred_element_type=f32)
        xps = [xp[80 * k:80 * k + CYC] for k in range(K)]
        CS = jnp.concatenate(
            [jnp.sum(s, axis=0, keepdims=True) for s in xps], axis=0)
        MU = jnp.dot(jnp.dot(CS, G), G.T) * (1.0 / 256.0)
        devs = [xps[k] - MU[k:k + 1] for k in range(K)]
        VS = jnp.concatenate(
            [jnp.sum(d * d, axis=0, keepdims=True) for d in devs], axis=0)
        VR = jnp.dot(jnp.dot(VS, G), G.T) * (1.0 / 256.0)
        UB = jnp.concatenate(
            [jnp.sum(U[CYC * k:CYC * (k + 1)], axis=0, keepdims=True)
             for k in range(K)], axis=0) * (1.0 / 64.0)
        GV = jax.nn.sigmoid(jnp.dot(UB, W_gateT))
        xks = []
        for k in range(K):
            xn = devs[k] * jax.lax.rsqrt(VR[k:k + 1] + EPS_GN)
            xn = xn * gn_gamma + gn_beta
            xg = _gelu(xn)
            xks.append(xg * (GV[k:k + 1] * (ws_n[n][k] * res_row)))
        X_n.append(jnp.concatenate(xks, axis=0))

    for n in range(NB):
        periodic = A[n] + jax.lax.dot_general(
            M_n[n], X_n[n], (((0,), (0,)), ((), ())),
            preferred_element_type=f32)
        local = local_raw[n]
        num = jnp.sum(local * periodic, axis=0, keepdims=True)
        den = jnp.sum(periodic * periodic, axis=0, keepdims=True) + 1e-6
        local = local - (num / den) * periodic
        gate = gate_n[n]
        fused = gate * periodic + (1.0 - gate) * 0.3 * local
        out_ref[n] = A[n] + fused


@jax.jit
def kernel(x, W_dw, W_pw, gn_gamma, gn_beta, W_gate, res_gamma, W_loc_dw,
           W_loc_pw):
    B = x.shape[0]
    W_dw9 = W_dw[:, 0, :].T
    W_locj = (W_loc_dw[:, 0, :].T[:, :, None]
              * W_loc_pw.T[None, :, :]).reshape(7 * C, C).astype(jnp.bfloat16)
    res_row = res_gamma[:, :, 0, 0]
    gng = gn_gamma[None, :]
    gnb = gn_beta[None, :]

    full = lambda shape: pl.BlockSpec(shape, lambda b: (0,) * len(shape))
    return pl.pallas_call(
        _body,
        grid=(B // NB,),
        in_specs=[
            pl.BlockSpec((NB, T, C), lambda b: (b, 0, 0)),
            full((9, C)), full((C, C)), full((1, C)), full((1, C)),
            full((C, C)), full((1, C)), full((7 * C, C)),
        ],
        out_specs=pl.BlockSpec((NB, T, C), lambda b: (b, 0, 0)),
        out_shape=jax.ShapeDtypeStruct((B, T, C), jnp.float32),
        compiler_params=pltpu.CompilerParams(
            dimension_semantics=("parallel",),
            fuse_transposed_lhs_in_matmul=True),
    )(x, W_dw9, W_pw.T, gng, gnb, W_gate.T, res_row, W_locj)
